# speedup vs baseline: 10.0694x; 10.0694x over previous
import functools

import jax
import jax.numpy as jnp
from jax import lax
from jax.experimental import pallas as pl
from jax.experimental.pallas import tpu as pltpu
from jax.experimental.pallas import tpu_sc as plsc

N = 10000
E = 320000
G = 128
NP = 10240
RPT = NP // 16
EP = 323584
K = 128
EPT = EP // 16
NCH = EPT // K
EPT_A = EP // 32
NCH_A = EPT_A // K
PAD_E = EP - E
PAD_ROWS = NP - N

_mesh = plsc.VectorSubcoreMesh(core_axis_name="c", subcore_axis_name="s")


@functools.partial(
    pl.kernel,
    mesh=_mesh,
    out_type=jax.ShapeDtypeStruct((2 * NP,), jnp.float32),
    scratch_types=[
        pltpu.VMEM((1, K), jnp.int32),
        pltpu.VMEM((K,), jnp.float32),
        pltpu.VMEM((RPT,), jnp.float32),
        pltpu.VMEM_SHARED((NP,), jnp.float32),
    ],
)
def _degree_kernel(col_hbm, deg_hbm, idx_v, ones_v, zeros_v, deg_sh):
    cid = lax.axis_index("c")
    tid = lax.axis_index("s")
    for j in range(RPT // 16):
        zeros_v[pl.ds(j * 16, 16)] = jnp.zeros((16,), jnp.float32)
    for j in range(K // 16):
        ones_v[pl.ds(j * 16, 16)] = jnp.full((16,), 1.0, jnp.float32)
    rbase = tid * RPT
    pltpu.sync_copy(zeros_v, deg_sh.at[pl.ds(rbase, RPT)])
    plsc.subcore_barrier()
    ebase = cid * (EP // 2) + tid * EPT_A

    def step(j, carry):
        off = ebase + j * K
        pltpu.sync_copy(col_hbm.at[pl.ds(off, K)], idx_v.at[0])
        pltpu.sync_copy(ones_v, deg_sh.at[idx_v.at[0]], add=True)
        return carry

    lax.fori_loop(0, NCH_A, step, 0)
    plsc.subcore_barrier()
    pltpu.sync_copy(deg_sh.at[pl.ds(rbase, RPT)],
                    deg_hbm.at[pl.ds(cid * NP + rbase, RPT)])


@functools.partial(
    pl.kernel,
    mesh=_mesh,
    out_type=jax.ShapeDtypeStruct((2 * NP, 128), jnp.float32),
    scratch_types=[
        pltpu.VMEM((1, K), jnp.int32),
        pltpu.VMEM((1, K), jnp.int32),
        pltpu.VMEM((1, K, 128), jnp.float32),
        pltpu.VMEM_SHARED((NP, 128), jnp.float32),
        pltpu.SemaphoreType.DMA,
    ],
)
def _propagate_kernel(g_hbm, row_hbm, col_hbm, acc_hbm,
                      idxr, idxc, rows, acc_sh, sem):
    cid = lax.axis_index("c")
    tid = lax.axis_index("s")
    rbase = tid * RPT
    pltpu.sync_copy(g_hbm.at[pl.ds(cid * NP + rbase, RPT)],
                    acc_sh.at[pl.ds(rbase, RPT)])
    plsc.subcore_barrier()
    ebase = tid * EPT

    def step(j, carry):
        off = ebase + j * K
        pltpu.sync_copy(row_hbm.at[pl.ds(cid * EP + off, K)], idxr.at[0])
        pltpu.sync_copy(col_hbm.at[pl.ds(off, K)], idxc.at[0])
        pltpu.async_copy(g_hbm.at[idxr.at[0]], rows.at[0], sem).wait()
        pltpu.sync_copy(rows.at[0], acc_sh.at[idxc.at[0]], add=True)
        return carry

    lax.fori_loop(0, NCH, step, 0)
    plsc.subcore_barrier()
    pltpu.sync_copy(acc_sh.at[pl.ds(rbase, RPT)],
                    acc_hbm.at[pl.ds(cid * NP + rbase, RPT)])


BLK = 1024
NSTEP = NP // BLK


def _dinv_of(deg_ref):
    deg = deg_ref[0] + deg_ref[1] + 1.0
    return lax.rsqrt(jnp.maximum(deg, 1e-12))


def _b1_body(x_ref, deg_ref, w_ref, g_ref):
    dinv = _dinv_of(deg_ref)
    h = jnp.dot(x_ref[...], w_ref[...], preferred_element_type=jnp.float32)
    g = h * dinv[:, None]
    g_ref[0] = g[:, :128]
    g_ref[1] = g[:, 128:]


def _layer_body(acc_ref, deg_ref, w_ref, b_ref, g_ref):
    dinv = _dinv_of(deg_ref)
    acc = jnp.concatenate([acc_ref[0], acc_ref[1]], axis=1)
    t = jnp.maximum(acc * dinv[:, None] + b_ref[...], 0.0)
    g = jnp.dot(t, w_ref[...], preferred_element_type=jnp.float32)
    g = g * dinv[:, None]
    g_ref[0] = g[:, :128]
    g_ref[1] = g[:, 128:]


def _pool_body(acc_ref, deg_ref, batch_ref, b3_ref, wm1_ref, bm1_ref,
               wm2_ref, bm2_ref, out_ref, pooled_acc, cnt_acc):
    i = pl.program_id(0)

    @pl.when(i == 0)
    def _():
        pooled_acc[...] = jnp.zeros_like(pooled_acc)
        cnt_acc[...] = jnp.zeros_like(cnt_acc)

    dinv = _dinv_of(deg_ref)
    h3 = jnp.concatenate([acc_ref[0], acc_ref[1]], axis=1) * dinv[:, None]
    gids = lax.broadcasted_iota(jnp.int32, (G, BLK), 0)
    onehot = (batch_ref[0][None, :] == gids).astype(jnp.float32)
    pooled_acc[...] += jnp.dot(onehot, h3, preferred_element_type=jnp.float32)
    cnt_acc[...] += jnp.dot(onehot, jnp.ones((BLK, 128), jnp.float32),
                            preferred_element_type=jnp.float32)

    @pl.when(i == NSTEP - 1)
    def _():
        cnt = jnp.maximum(cnt_acc[...], 1.0)
        pm = pooled_acc[...] / jnp.concatenate([cnt, cnt], axis=1) + b3_ref[...]
        h = jnp.maximum(
            jnp.dot(pm, wm1_ref[...], preferred_element_type=jnp.float32)
            + bm1_ref[...], 0.0)
        out_ref[...] = (
            jnp.dot(h, wm2_ref[...], preferred_element_type=jnp.float32)
            + bm2_ref[...])


def _b1(x_p, deg2, W1):
    return pl.pallas_call(
        _b1_body,
        grid=(NSTEP,),
        in_specs=[
            pl.BlockSpec((BLK, 128), lambda i: (i, 0)),
            pl.BlockSpec((2, BLK), lambda i: (0, i)),
            pl.BlockSpec((128, 256), lambda i: (0, 0)),
        ],
        out_specs=pl.BlockSpec((2, BLK, 128), lambda i: (0, i, 0)),
        out_shape=jax.ShapeDtypeStruct((2, NP, 128), jnp.float32),
    )(x_p, deg2, W1)


def _layer(acc, deg2, W, b):
    return pl.pallas_call(
        _layer_body,
        grid=(NSTEP,),
        in_specs=[
            pl.BlockSpec((2, BLK, 128), lambda i: (0, i, 0)),
            pl.BlockSpec((2, BLK), lambda i: (0, i)),
            pl.BlockSpec((256, 256), lambda i: (0, 0)),
            pl.BlockSpec((1, 256), lambda i: (0, 0)),
        ],
        out_specs=pl.BlockSpec((2, BLK, 128), lambda i: (0, i, 0)),
        out_shape=jax.ShapeDtypeStruct((2, NP, 128), jnp.float32),
    )(acc, deg2, W, b)


def _pool_mlp(acc, deg2, batch_p, b3, Wm1, bm1, Wm2, bm2):
    return pl.pallas_call(
        _pool_body,
        grid=(NSTEP,),
        in_specs=[
            pl.BlockSpec((2, BLK, 128), lambda i: (0, i, 0)),
            pl.BlockSpec((2, BLK), lambda i: (0, i)),
            pl.BlockSpec((1, BLK), lambda i: (0, i)),
            pl.BlockSpec((1, 256), lambda i: (0, 0)),
            pl.BlockSpec((256, 256), lambda i: (0, 0)),
            pl.BlockSpec((1, 256), lambda i: (0, 0)),
            pl.BlockSpec((256, 128), lambda i: (0, 0)),
            pl.BlockSpec((1, 128), lambda i: (0, 0)),
        ],
        out_specs=pl.BlockSpec((G, 128), lambda i: (0, 0)),
        out_shape=jax.ShapeDtypeStruct((G, 128), jnp.float32),
        scratch_shapes=[
            pltpu.VMEM((G, 256), jnp.float32),
            pltpu.VMEM((G, 128), jnp.float32),
        ],
    )(acc, deg2, batch_p, b3, Wm1, bm1, Wm2, bm2)


def kernel(x, edge_index, batch, W1, b1, W2, b2, W3, b3, Wm1, bm1, Wm2, bm2):
    row = edge_index[0].astype(jnp.int32)
    col = edge_index[1].astype(jnp.int32)
    pad_ids = N + (jnp.arange(PAD_E, dtype=jnp.int32) % PAD_ROWS)
    row_p = jnp.concatenate([row, pad_ids])
    col_p = jnp.concatenate([col, pad_ids])
    row2 = jnp.concatenate([row_p, row_p + NP])
    x_p = jnp.pad(x, ((0, PAD_ROWS), (0, 0)))
    batch_p = jnp.concatenate(
        [batch.astype(jnp.int32), jnp.full((PAD_ROWS,), G, jnp.int32)]
    )[None, :]
    b1r, b2r, b3r = b1[None, :], b2[None, :], b3[None, :]
    bm1r, bm2r = bm1[None, :], bm2[None, :]

    deg = _degree_kernel(col_p)
    deg2 = deg.reshape(2, NP)

    g1 = _b1(x_p, deg2, W1).reshape(2 * NP, 128)
    acc1 = _propagate_kernel(g1, row2, col_p).reshape(2, NP, 128)
    g2 = _layer(acc1, deg2, W2, b1r).reshape(2 * NP, 128)
    acc2 = _propagate_kernel(g2, row2, col_p).reshape(2, NP, 128)
    g3 = _layer(acc2, deg2, W3, b2r).reshape(2 * NP, 128)
    acc3 = _propagate_kernel(g3, row2, col_p).reshape(2, NP, 128)

    return _pool_mlp(acc3, deg2, batch_p, b3r, Wm1, bm1r, Wm2, bm2r)


# speedup vs baseline: 19.1961x; 1.9064x over previous
import functools

import jax
import jax.numpy as jnp
from jax import lax
from jax.experimental import pallas as pl
from jax.experimental.pallas import tpu as pltpu
from jax.experimental.pallas import tpu_sc as plsc

N = 10000
E = 320000
G = 128
NP = 10240
RPT = NP // 16
EP = 323584
K = 128
EPT = EP // 16
NCH = EPT // K
EPT_A = EP // 32
NCH_A = EPT_A // K
PAD_E = EP - E
PAD_ROWS = NP - N

_mesh = plsc.VectorSubcoreMesh(core_axis_name="c", subcore_axis_name="s")


@functools.partial(
    pl.kernel,
    mesh=_mesh,
    out_type=jax.ShapeDtypeStruct((2 * NP,), jnp.float32),
    scratch_types=[
        pltpu.VMEM((NCH_A, K), jnp.int32),
        pltpu.VMEM((K,), jnp.float32),
        pltpu.VMEM((RPT,), jnp.float32),
        pltpu.VMEM_SHARED((NP,), jnp.float32),
    ],
)
def _degree_kernel(col_hbm, deg_hbm, idx_v, ones_v, zeros_v, deg_sh):
    cid = lax.axis_index("c")
    tid = lax.axis_index("s")
    for j in range(RPT // 16):
        zeros_v[pl.ds(j * 16, 16)] = jnp.zeros((16,), jnp.float32)
    for j in range(K // 16):
        ones_v[pl.ds(j * 16, 16)] = jnp.full((16,), 1.0, jnp.float32)
    rbase = tid * RPT
    pltpu.sync_copy(zeros_v, deg_sh.at[pl.ds(rbase, RPT)])
    pltpu.sync_copy(col_hbm.at[cid, tid], idx_v)
    plsc.subcore_barrier()

    def step(j, carry):
        pltpu.sync_copy(ones_v, deg_sh.at[idx_v.at[j]], add=True)
        return carry

    lax.fori_loop(0, NCH_A, step, 0)
    plsc.subcore_barrier()
    pltpu.sync_copy(deg_sh.at[pl.ds(rbase, RPT)],
                    deg_hbm.at[pl.ds(cid * NP + rbase, RPT)])


@functools.partial(
    pl.kernel,
    mesh=_mesh,
    out_type=jax.ShapeDtypeStruct((2 * NP, 128), jnp.float32),
    scratch_types=[
        pltpu.VMEM((2, 2, 2, K), jnp.int32),
        pltpu.VMEM((2, K, 128), jnp.float32),
        pltpu.VMEM_SHARED((NP, 128), jnp.float32),
        pltpu.SemaphoreType.DMA,
        pltpu.SemaphoreType.DMA,
    ],
)
def _propagate_kernel(g_hbm, rc_hbm, acc_hbm, idx, rows, acc_sh, semg, semi):
    cid = lax.axis_index("c")
    tid = lax.axis_index("s")
    rbase = tid * RPT
    pltpu.sync_copy(g_hbm.at[pl.ds(cid * NP + rbase, RPT)],
                    acc_sh.at[pl.ds(rbase, RPT)])
    pltpu.sync_copy(rc_hbm.at[cid, tid, pl.ds(0, 2)], idx.at[0])
    plsc.subcore_barrier()

    def wait_gather():
        pltpu.make_async_copy(g_hbm.at[pl.ds(0, K)], rows.at[0], semg).wait()

    pltpu.async_copy(g_hbm.at[idx.at[0, 0, 0]], rows.at[0], semg)

    def body(jj, carry):
        p = jj % 2
        jn = jnp.minimum(2 * jj + 2, NCH - 2)
        wait_gather()
        pltpu.async_copy(g_hbm.at[idx.at[p, 1, 0]], rows.at[1], semg)
        pltpu.sync_copy(rows.at[0], acc_sh.at[idx.at[p, 0, 1]], add=True)
        pltpu.async_copy(rc_hbm.at[cid, tid, pl.ds(jn, 2)], idx.at[1 - p], semi)
        wait_gather()
        pltpu.make_async_copy(
            rc_hbm.at[cid, tid, pl.ds(0, 2)], idx.at[0], semi).wait()
        pltpu.async_copy(g_hbm.at[idx.at[1 - p, 0, 0]], rows.at[0], semg)
        pltpu.sync_copy(rows.at[1], acc_sh.at[idx.at[p, 1, 1]], add=True)
        return carry

    lax.fori_loop(0, NCH // 2, body, 0)
    wait_gather()
    plsc.subcore_barrier()
    pltpu.sync_copy(acc_sh.at[pl.ds(rbase, RPT)],
                    acc_hbm.at[pl.ds(cid * NP + rbase, RPT)])


BLK = 1024
NSTEP = NP // BLK


def _dinv_of(deg_ref):
    deg = deg_ref[0] + deg_ref[1] + 1.0
    return lax.rsqrt(jnp.maximum(deg, 1e-12))


def _b1_body(x_ref, deg_ref, w_ref, g_ref):
    dinv = _dinv_of(deg_ref)
    h = jnp.dot(x_ref[...], w_ref[...], preferred_element_type=jnp.float32)
    g = h * dinv[:, None]
    g_ref[0] = g[:, :128]
    g_ref[1] = g[:, 128:]


def _layer_body(acc_ref, deg_ref, w_ref, b_ref, g_ref):
    dinv = _dinv_of(deg_ref)
    acc = jnp.concatenate([acc_ref[0], acc_ref[1]], axis=1)
    t = jnp.maximum(acc * dinv[:, None] + b_ref[...], 0.0)
    g = jnp.dot(t, w_ref[...], preferred_element_type=jnp.float32)
    g = g * dinv[:, None]
    g_ref[0] = g[:, :128]
    g_ref[1] = g[:, 128:]


def _pool_body(acc_ref, deg_ref, batch_ref, b3_ref, wm1_ref, bm1_ref,
               wm2_ref, bm2_ref, out_ref, pooled_acc, cnt_acc):
    i = pl.program_id(0)

    @pl.when(i == 0)
    def _():
        pooled_acc[...] = jnp.zeros_like(pooled_acc)
        cnt_acc[...] = jnp.zeros_like(cnt_acc)

    dinv = _dinv_of(deg_ref)
    h3 = jnp.concatenate([acc_ref[0], acc_ref[1]], axis=1) * dinv[:, None]
    gids = lax.broadcasted_iota(jnp.int32, (G, BLK), 0)
    onehot = (batch_ref[0][None, :] == gids).astype(jnp.float32)
    pooled_acc[...] += jnp.dot(onehot, h3, preferred_element_type=jnp.float32)
    cnt_acc[...] += jnp.dot(onehot, jnp.ones((BLK, 128), jnp.float32),
                            preferred_element_type=jnp.float32)

    @pl.when(i == NSTEP - 1)
    def _():
        cnt = jnp.maximum(cnt_acc[...], 1.0)
        pm = pooled_acc[...] / jnp.concatenate([cnt, cnt], axis=1) + b3_ref[...]
        h = jnp.maximum(
            jnp.dot(pm, wm1_ref[...], preferred_element_type=jnp.float32)
            + bm1_ref[...], 0.0)
        out_ref[...] = (
            jnp.dot(h, wm2_ref[...], preferred_element_type=jnp.float32)
            + bm2_ref[...])


def _b1(x_p, deg2, W1):
    return pl.pallas_call(
        _b1_body,
        grid=(NSTEP,),
        in_specs=[
            pl.BlockSpec((BLK, 128), lambda i: (i, 0)),
            pl.BlockSpec((2, BLK), lambda i: (0, i)),
            pl.BlockSpec((128, 256), lambda i: (0, 0)),
        ],
        out_specs=pl.BlockSpec((2, BLK, 128), lambda i: (0, i, 0)),
        out_shape=jax.ShapeDtypeStruct((2, NP, 128), jnp.float32),
    )(x_p, deg2, W1)


def _layer(acc, deg2, W, b):
    return pl.pallas_call(
        _layer_body,
        grid=(NSTEP,),
        in_specs=[
            pl.BlockSpec((2, BLK, 128), lambda i: (0, i, 0)),
            pl.BlockSpec((2, BLK), lambda i: (0, i)),
            pl.BlockSpec((256, 256), lambda i: (0, 0)),
            pl.BlockSpec((1, 256), lambda i: (0, 0)),
        ],
        out_specs=pl.BlockSpec((2, BLK, 128), lambda i: (0, i, 0)),
        out_shape=jax.ShapeDtypeStruct((2, NP, 128), jnp.float32),
    )(acc, deg2, W, b)


def _pool_mlp(acc, deg2, batch_p, b3, Wm1, bm1, Wm2, bm2):
    return pl.pallas_call(
        _pool_body,
        grid=(NSTEP,),
        in_specs=[
            pl.BlockSpec((2, BLK, 128), lambda i: (0, i, 0)),
            pl.BlockSpec((2, BLK), lambda i: (0, i)),
            pl.BlockSpec((1, BLK), lambda i: (0, i)),
            pl.BlockSpec((1, 256), lambda i: (0, 0)),
            pl.BlockSpec((256, 256), lambda i: (0, 0)),
            pl.BlockSpec((1, 256), lambda i: (0, 0)),
            pl.BlockSpec((256, 128), lambda i: (0, 0)),
            pl.BlockSpec((1, 128), lambda i: (0, 0)),
        ],
        out_specs=pl.BlockSpec((G, 128), lambda i: (0, 0)),
        out_shape=jax.ShapeDtypeStruct((G, 128), jnp.float32),
        scratch_shapes=[
            pltpu.VMEM((G, 256), jnp.float32),
            pltpu.VMEM((G, 128), jnp.float32),
        ],
    )(acc, deg2, batch_p, b3, Wm1, bm1, Wm2, bm2)


def kernel(x, edge_index, batch, W1, b1, W2, b2, W3, b3, Wm1, bm1, Wm2, bm2):
    row = edge_index[0].astype(jnp.int32)
    col = edge_index[1].astype(jnp.int32)
    pad_ids = N + (jnp.arange(PAD_E, dtype=jnp.int32) % PAD_ROWS)
    row_p = jnp.concatenate([row, pad_ids])
    col_p = jnp.concatenate([col, pad_ids])
    row2 = jnp.concatenate([row_p, row_p + NP])
    x_p = jnp.pad(x, ((0, PAD_ROWS), (0, 0)))
    batch_p = jnp.concatenate(
        [batch.astype(jnp.int32), jnp.full((PAD_ROWS,), G, jnp.int32)]
    )[None, :]
    b1r, b2r, b3r = b1[None, :], b2[None, :], b3[None, :]
    bm1r, bm2r = bm1[None, :], bm2[None, :]

    col_a = col_p.reshape(2, 16, NCH_A, K)
    row_c = row2.reshape(2, 16, NCH, 1, K)
    col_c = jnp.broadcast_to(col_p.reshape(1, 16, NCH, 1, K),
                             (2, 16, NCH, 1, K))
    rc = jnp.concatenate([row_c, col_c], axis=3)

    deg = _degree_kernel(col_a)
    deg2 = deg.reshape(2, NP)

    g1 = _b1(x_p, deg2, W1).reshape(2 * NP, 128)
    acc1 = _propagate_kernel(g1, rc).reshape(2, NP, 128)
    g2 = _layer(acc1, deg2, W2, b1r).reshape(2 * NP, 128)
    acc2 = _propagate_kernel(g2, rc).reshape(2, NP, 128)
    g3 = _layer(acc2, deg2, W3, b2r).reshape(2 * NP, 128)
    acc3 = _propagate_kernel(g3, rc).reshape(2, NP, 128)

    return _pool_mlp(acc3, deg2, batch_p, b3r, Wm1, bm1r, Wm2, bm2r)
